# Initial kernel scaffold; baseline (speedup 1.0000x reference)
import jax
import jax.numpy as jnp
from jax.experimental import pallas as pl


def kernel(x, lag_embed, val_w, val_b, conv_w, conv_b, ctxp_w, ctxp_b, wq, wk, bias_w, bias_b):
    raise NotImplementedError("write your pallas kernel here")


# speedup vs baseline: 8.6691x; 8.6691x over previous
import jax
import jax.numpy as jnp
from jax import lax
from jax.experimental import pallas as pl
from jax.experimental.pallas import tpu as pltpu

_L = 256
_D = 32
_TOPK = 16
_TT = 256
_NS = 32
_SEG = 8


def _dgd(a, b, dims=((1,), (0,))):
    return lax.dot_general(a, b, ((dims), ((), ())),
                           precision=lax.Precision.DEFAULT,
                           preferred_element_type=jnp.float32)


def _bf(a):
    return a.astype(jnp.bfloat16).astype(jnp.float32)


def _body(xf, xr, c8_r, vwc_r, cw_r, cb_r, cpw_r, cpb_r, wq_r, wk_r,
          bw_r, bb_r, mu_ref, w_ref):
    j = pl.program_id(1)
    t0 = j * _TT

    c8 = c8_r[:, :]
    vwc, cw, cb = vwc_r[:, :], cw_r[:, :], cb_r[:, :]
    cpw, cpb = cpw_r[:, :], cpb_r[:, :]
    wq_, wk_, bw, bb = wq_r[:, :], wk_r[:, :], bw_r[:, :], bb_r[:, :]

    xw = xf[0, :, pl.ds(t0, _TT + _L)]
    xrw = xr[0, :, pl.ds(2048 - t0, _TT + _L)]

    xwb = _bf(xw)
    cwb = _bf(cw)
    acc = jnp.broadcast_to(cb, (_D, _TT))
    for k in range(9):
        acc = acc + cwb[:, k][:, None] * xwb[:, _L - 8 + k: _L - 8 + k + _TT]
    ctx2 = _dgd(cpw, acc) + cpb
    qT = _dgd(wq_, ctx2)
    cT = _dgd(bw, ctx2) + bb

    qts = [jnp.transpose(qT[:, 32 * s: 32 * s + 32]) for s in range(_SEG)]

    lane = lax.broadcasted_iota(jnp.int32, (_SEG, _SEG * _L), 1)
    sub = lax.broadcasted_iota(jnp.int32, (_SEG, _SEG * _L), 0)
    m8 = ((lane >> 8) == sub).astype(jnp.float32)

    rows = []
    for sb in range(_NS):
        xsegs = [xrw[:, 255 - sb - 32 * s: 511 - sb - 32 * s]
                 for s in range(_SEG)]
        xrow = jnp.concatenate(xsegs, axis=1)
        hb = vwc * xrow + c8
        kb = _dgd(wk_, hb)
        q8 = jnp.concatenate([qts[s][sb:sb + 1, :] for s in range(_SEG)],
                             axis=0)
        l8 = _dgd(q8, kb)
        rows.append(jnp.sum(l8 * m8, axis=0, keepdims=True))
    lg = jnp.concatenate(rows, axis=0)
    lt = jnp.concatenate(
        [jnp.transpose(lg[:, _L * s: _L * (s + 1)]) for s in range(_SEG)],
        axis=1)

    base = jnp.broadcast_to(xw, (_L, _TT + _L))
    yt = pltpu.roll(base, _TT + 1, 1, stride=1, stride_axis=0)[:, :_TT]

    run = lt
    v1 = jnp.max(run, axis=0, keepdims=True)
    vk = v1
    for _ in range(_TOPK - 1):
        run = jnp.where(run >= vk, -jnp.inf, run)
        vk = jnp.max(run, axis=0, keepdims=True)

    mask = lt >= vk
    ex = jnp.where(mask, jnp.exp(lt - v1), 0.0)
    s = jnp.sum(ex, axis=0, keepdims=True)
    wT = ex / s

    mu_ar = jnp.sum(wT * yt, axis=0, keepdims=True)
    mu_ref[0, :, :] = mu_ar + cT
    w_ref[0, :, :] = wT.T


@jax.jit
def kernel(x, lag_embed, val_w, val_b, conv_w, conv_b, ctxp_w, ctxp_b,
           wq, wk, bias_w, bias_b):
    B, T = x.shape

    xf = jnp.pad(x, ((0, 0), (_L, 0)))
    xr = jnp.pad(xf[:, ::-1], ((0, 0), (_L - 1, 1)))
    xf = xf.reshape(B, 1, _L + T)
    xr = xr.reshape(B, 1, 2 * _L + T)

    ec = (lag_embed[1:_L + 1] + val_b[None, :]).T
    c8 = jnp.tile(ec, (1, _SEG))
    vwc = val_w.reshape(_D, 1)
    cw = conv_w.reshape(_D, 9)
    cb = conv_b.reshape(_D, 1)
    cpb = ctxp_b.reshape(_D, 1)
    bb = bias_b.reshape(1, 1)

    nj = T // _TT
    const = lambda shape: pl.BlockSpec(shape, lambda b, j: (0, 0))
    grid_spec = pl.GridSpec(
        grid=(B, nj),
        in_specs=[
            pl.BlockSpec((1, 1, _L + T), lambda b, j: (b, 0, 0)),
            pl.BlockSpec((1, 1, 2 * _L + T), lambda b, j: (b, 0, 0)),
            const((_D, _SEG * _L)),
            const((_D, 1)),
            const((_D, 9)),
            const((_D, 1)),
            const((_D, _D)),
            const((_D, 1)),
            const((_D, _D)),
            const((_D, _D)),
            const((1, _D)),
            const((1, 1)),
        ],
        out_specs=[
            pl.BlockSpec((1, 1, _TT), lambda b, j: (b, 0, j)),
            pl.BlockSpec((1, _TT, _L), lambda b, j: (b, j, 0)),
        ],
    )
    mu, w = pl.pallas_call(
        _body,
        grid_spec=grid_spec,
        out_shape=[
            jax.ShapeDtypeStruct((B, 1, T), jnp.float32),
            jax.ShapeDtypeStruct((B, T, _L), jnp.float32),
        ],
        compiler_params=pltpu.CompilerParams(
            dimension_semantics=("parallel", "arbitrary"),
        ),
    )(xf, xr, c8, vwc, cw, cb, ctxp_w, cpb, wq, wk, bias_w, bb)
    return (mu.reshape(B, T), w)


# speedup vs baseline: 13.6246x; 1.5716x over previous
import jax
import jax.numpy as jnp
from jax import lax
from jax.experimental import pallas as pl
from jax.experimental.pallas import tpu as pltpu

_L = 256
_D = 32
_TOPK = 16
_TT = 256
_NS = 32
_SEG = 8


def _dgd(a, b, dims=((1,), (0,))):
    return lax.dot_general(a, b, ((dims), ((), ())),
                           precision=lax.Precision.DEFAULT,
                           preferred_element_type=jnp.float32)


def _bf(a):
    return a.astype(jnp.bfloat16).astype(jnp.float32)


def _body(xf, xr, c8_r, vwc_r, cw_r, cb_r, cpw_r, cpb_r, wq_r, wk_r,
          bw_r, bb_r, mu_ref, w_ref):
    j = pl.program_id(1)
    t0 = j * _TT

    c8 = c8_r[:, :]
    vwc, cw, cb = vwc_r[:, :], cw_r[:, :], cb_r[:, :]
    cpw, cpb = cpw_r[:, :], cpb_r[:, :]
    wq_, wk_, bw, bb = wq_r[:, :], wk_r[:, :], bw_r[:, :], bb_r[:, :]

    xw = xf[0, :, pl.ds(t0, _TT + _L)]
    xrw = xr[0, :, pl.ds(2048 - t0, _TT + _L)]

    xwb = _bf(xw)
    cwb = _bf(cw)
    acc = jnp.broadcast_to(cb, (_D, _TT))
    for k in range(9):
        acc = acc + cwb[:, k][:, None] * xwb[:, _L - 8 + k: _L - 8 + k + _TT]
    ctx2 = _dgd(cpw, acc) + cpb
    qT = _dgd(wq_, ctx2)
    cT = _dgd(bw, ctx2) + bb

    qts = [jnp.transpose(qT[:, 32 * s: 32 * s + 32]) for s in range(_SEG)]

    lane = lax.broadcasted_iota(jnp.int32, (_SEG, _SEG * _L), 1)
    sub = lax.broadcasted_iota(jnp.int32, (_SEG, _SEG * _L), 0)
    m8 = ((lane >> 8) == sub).astype(jnp.float32)
    c84 = jnp.concatenate([c8] * 4, axis=1)

    rows = []
    for g in range(_NS // 4):
        xsegs = []
        for sb in range(4 * g, 4 * g + 4):
            xsegs += [xrw[:, 255 - sb - 32 * s: 511 - sb - 32 * s]
                      for s in range(_SEG)]
        xrow4 = jnp.concatenate(xsegs, axis=1)
        hb4 = vwc * xrow4 + c84
        kb4 = _dgd(wk_, hb4)
        for i, sb in enumerate(range(4 * g, 4 * g + 4)):
            kb = kb4[:, 2048 * i: 2048 * (i + 1)]
            q8 = jnp.concatenate(
                [qts[s][sb:sb + 1, :] for s in range(_SEG)], axis=0)
            l8 = _dgd(q8, kb)
            rows.append(jnp.sum(l8 * m8, axis=0, keepdims=True))
    lg = jnp.concatenate(rows, axis=0)
    lt = jnp.concatenate(
        [jnp.transpose(lg[:, _L * s: _L * (s + 1)]) for s in range(_SEG)],
        axis=1)

    base = jnp.broadcast_to(xw, (_L, _TT + _L))
    yt = pltpu.roll(base, _TT + 1, 1, stride=1, stride_axis=0)[:, :_TT]

    run = lt
    v1 = jnp.max(run, axis=0, keepdims=True)
    vk = v1
    for _ in range(_TOPK - 1):
        run = jnp.where(run >= vk, -jnp.inf, run)
        vk = jnp.max(run, axis=0, keepdims=True)

    mask = lt >= vk
    ex = jnp.where(mask, jnp.exp(lt - v1), 0.0)
    s = jnp.sum(ex, axis=0, keepdims=True)
    wT = ex / s

    mu_ar = jnp.sum(wT * yt, axis=0, keepdims=True)
    mu_ref[0, :, :] = mu_ar + cT
    w_ref[0, :, :] = wT.T


@jax.jit
def kernel(x, lag_embed, val_w, val_b, conv_w, conv_b, ctxp_w, ctxp_b,
           wq, wk, bias_w, bias_b):
    B, T = x.shape

    xf = jnp.pad(x, ((0, 0), (_L, 0)))
    xr = jnp.pad(xf[:, ::-1], ((0, 0), (_L - 1, 1)))
    xf = xf.reshape(B, 1, _L + T)
    xr = xr.reshape(B, 1, 2 * _L + T)

    ec = (lag_embed[1:_L + 1] + val_b[None, :]).T
    c8 = jnp.tile(ec, (1, _SEG))
    vwc = val_w.reshape(_D, 1)
    cw = conv_w.reshape(_D, 9)
    cb = conv_b.reshape(_D, 1)
    cpb = ctxp_b.reshape(_D, 1)
    bb = bias_b.reshape(1, 1)

    nj = T // _TT
    const = lambda shape: pl.BlockSpec(shape, lambda b, j: (0, 0))
    grid_spec = pl.GridSpec(
        grid=(B, nj),
        in_specs=[
            pl.BlockSpec((1, 1, _L + T), lambda b, j: (b, 0, 0)),
            pl.BlockSpec((1, 1, 2 * _L + T), lambda b, j: (b, 0, 0)),
            const((_D, _SEG * _L)),
            const((_D, 1)),
            const((_D, 9)),
            const((_D, 1)),
            const((_D, _D)),
            const((_D, 1)),
            const((_D, _D)),
            const((_D, _D)),
            const((1, _D)),
            const((1, 1)),
        ],
        out_specs=[
            pl.BlockSpec((1, 1, _TT), lambda b, j: (b, 0, j)),
            pl.BlockSpec((1, _TT, _L), lambda b, j: (b, j, 0)),
        ],
    )
    mu, w = pl.pallas_call(
        _body,
        grid_spec=grid_spec,
        out_shape=[
            jax.ShapeDtypeStruct((B, 1, T), jnp.float32),
            jax.ShapeDtypeStruct((B, T, _L), jnp.float32),
        ],
        compiler_params=pltpu.CompilerParams(
            dimension_semantics=("parallel", "arbitrary"),
        ),
    )(xf, xr, c8, vwc, cw, cb, ctxp_w, cpb, wq, wk, bias_w, bb)
    return (mu.reshape(B, T), w)


# speedup vs baseline: 13.9871x; 1.0266x over previous
import jax
import jax.numpy as jnp
from jax import lax
from jax.experimental import pallas as pl
from jax.experimental.pallas import tpu as pltpu

_L = 256
_D = 32
_TOPK = 16
_TT = 512
_SEG = 8
_NS = _TT // _SEG


def _dgd(a, b, dims=((1,), (0,))):
    return lax.dot_general(a, b, ((dims), ((), ())),
                           precision=lax.Precision.DEFAULT,
                           preferred_element_type=jnp.float32)


def _bf(a):
    return a.astype(jnp.bfloat16).astype(jnp.float32)


def _body(xf, xr, c8_r, vwc_r, cw_r, cb_r, cpw_r, cpb_r, wq_r, wk_r,
          bw_r, bb_r, mu_ref, w_ref):
    j = pl.program_id(1)
    t0 = j * _TT

    c8 = c8_r[:, :]
    vwc, cw, cb = vwc_r[:, :], cw_r[:, :], cb_r[:, :]
    cpw, cpb = cpw_r[:, :], cpb_r[:, :]
    wq_, wk_, bw, bb = wq_r[:, :], wk_r[:, :], bw_r[:, :], bb_r[:, :]

    xw = xf[0, :, pl.ds(t0, _TT + _L)]
    xrw = xr[0, :, pl.ds(2304 - _TT - t0, _TT + _L)]

    xwb = _bf(xw)
    cwb = _bf(cw)
    acc = jnp.broadcast_to(cb, (_D, _TT))
    for k in range(9):
        acc = acc + cwb[:, k][:, None] * xwb[:, _L - 8 + k: _L - 8 + k + _TT]
    ctx2 = _dgd(cpw, acc) + cpb
    qT = _dgd(wq_, ctx2)
    cT = _dgd(bw, ctx2) + bb

    qtok = jnp.transpose(qT)

    lane = lax.broadcasted_iota(jnp.int32, (_SEG, _SEG * _L), 1)
    sub = lax.broadcasted_iota(jnp.int32, (_SEG, _SEG * _L), 0)
    m8 = ((lane >> 8) == sub).astype(jnp.float32)
    c84 = jnp.concatenate([c8] * 4, axis=1)

    rows = []
    for g in range(_NS // 4):
        xsegs = []
        for sb in range(4 * g, 4 * g + 4):
            xsegs += [xrw[:, _TT - 1 - sb - _NS * s: _TT - 1 - sb - _NS * s + _L]
                      for s in range(_SEG)]
        xrow4 = jnp.concatenate(xsegs, axis=1)
        hb4 = vwc * xrow4 + c84
        kb4 = _dgd(wk_, hb4)
        for i, sb in enumerate(range(4 * g, 4 * g + 4)):
            kb = kb4[:, 2048 * i: 2048 * (i + 1)]
            q8 = jnp.concatenate(
                [qtok[sb + _NS * s: sb + _NS * s + 1, :] for s in range(_SEG)],
                axis=0)
            l8 = _dgd(q8, kb)
            rows.append(jnp.sum(l8 * m8, axis=0, keepdims=True))
    lg = jnp.concatenate(rows, axis=0)
    lt = jnp.concatenate(
        [jnp.transpose(lg[:, _L * s: _L * (s + 1)]) for s in range(_SEG)],
        axis=1)

    base = jnp.broadcast_to(xw, (_L, _TT + _L))
    yt = pltpu.roll(base, _TT + 1, 1, stride=1, stride_axis=0)[:, :_TT]

    run = lt
    v1 = jnp.max(run, axis=0, keepdims=True)
    vk = v1
    for _ in range(_TOPK - 1):
        run = jnp.where(run >= vk, -jnp.inf, run)
        vk = jnp.max(run, axis=0, keepdims=True)

    mask = lt >= vk
    ex = jnp.where(mask, jnp.exp(lt - v1), 0.0)
    s = jnp.sum(ex, axis=0, keepdims=True)
    wT = ex / s

    mu_ar = jnp.sum(wT * yt, axis=0, keepdims=True)
    mu_ref[0, :, :] = mu_ar + cT
    w_ref[0, :, :] = wT.T


@jax.jit
def kernel(x, lag_embed, val_w, val_b, conv_w, conv_b, ctxp_w, ctxp_b,
           wq, wk, bias_w, bias_b):
    B, T = x.shape

    xf = jnp.pad(x, ((0, 0), (_L, 0)))
    xr = jnp.pad(xf[:, ::-1], ((0, 0), (_L - 1, 1)))
    xf = xf.reshape(B, 1, _L + T)
    xr = xr.reshape(B, 1, 2 * _L + T)

    ec = (lag_embed[1:_L + 1] + val_b[None, :]).T
    c8 = jnp.tile(ec, (1, _SEG))
    vwc = val_w.reshape(_D, 1)
    cw = conv_w.reshape(_D, 9)
    cb = conv_b.reshape(_D, 1)
    cpb = ctxp_b.reshape(_D, 1)
    bb = bias_b.reshape(1, 1)

    nj = T // _TT
    const = lambda shape: pl.BlockSpec(shape, lambda b, j: (0, 0))
    grid_spec = pl.GridSpec(
        grid=(B, nj),
        in_specs=[
            pl.BlockSpec((1, 1, _L + T), lambda b, j: (b, 0, 0)),
            pl.BlockSpec((1, 1, 2 * _L + T), lambda b, j: (b, 0, 0)),
            const((_D, _SEG * _L)),
            const((_D, 1)),
            const((_D, 9)),
            const((_D, 1)),
            const((_D, _D)),
            const((_D, 1)),
            const((_D, _D)),
            const((_D, _D)),
            const((1, _D)),
            const((1, 1)),
        ],
        out_specs=[
            pl.BlockSpec((1, 1, _TT), lambda b, j: (b, 0, j)),
            pl.BlockSpec((1, _TT, _L), lambda b, j: (b, j, 0)),
        ],
    )
    mu, w = pl.pallas_call(
        _body,
        grid_spec=grid_spec,
        out_shape=[
            jax.ShapeDtypeStruct((B, 1, T), jnp.float32),
            jax.ShapeDtypeStruct((B, T, _L), jnp.float32),
        ],
        compiler_params=pltpu.CompilerParams(
            dimension_semantics=("parallel", "arbitrary"),
        ),
    )(xf, xr, c8, vwc, cw, cb, ctxp_w, cpb, wq, wk, bias_w, bb)
    return (mu.reshape(B, T), w)


# speedup vs baseline: 15.9390x; 1.1396x over previous
import jax
import jax.numpy as jnp
from jax import lax
from jax.experimental import pallas as pl
from jax.experimental.pallas import tpu as pltpu

_L = 256
_D = 32
_TOPK = 16
_TT = 512
_SEG = 8
_NS = _TT // _SEG


def _dgd(a, b, dims=((1,), (0,))):
    return lax.dot_general(a, b, ((dims), ((), ())),
                           precision=lax.Precision.DEFAULT,
                           preferred_element_type=jnp.float32)


def _bf(a):
    return a.astype(jnp.bfloat16).astype(jnp.float32)


def _body(xf, xr, c84_r, m8_r, vwc_r, cw_r, cb_r, cpw_r, cpb_r, wq_r, wk_r,
          bw_r, bb_r, mu_ref, w_ref):
    j = pl.program_id(1)
    t0 = j * _TT

    c84 = c84_r[:, :]
    m8 = m8_r[:, :]
    vwc, cw, cb = vwc_r[:, :], cw_r[:, :], cb_r[:, :]
    cpw, cpb = cpw_r[:, :], cpb_r[:, :]
    wq_, wk_, bw, bb = wq_r[:, :], wk_r[:, :], bw_r[:, :], bb_r[:, :]

    xw = xf[0, :, pl.ds(t0, _TT + _L)]
    xrw = xr[0, :, pl.ds(2304 - _TT - t0, _TT + _L)]

    xwb = _bf(xw)
    cwb = _bf(cw)
    acc = jnp.broadcast_to(cb, (_D, _TT))
    for k in range(9):
        acc = acc + cwb[:, k][:, None] * xwb[:, _L - 8 + k: _L - 8 + k + _TT]
    ctx2 = _dgd(cpw, acc) + cpb
    qT = _dgd(wq_, ctx2)
    cT = _dgd(bw, ctx2) + bb

    qtok = jnp.transpose(qT)

    base = jnp.broadcast_to(xw, (_L, _TT + _L))
    yt = pltpu.roll(base, _TT + 1, 1, stride=1, stride_axis=0)[:, :_TT]

    mu_parts = []
    for c in range(_TT // 128):
        rows = []
        for gg in range(4):
            xsegs = []
            for sbl in range(4 * gg, 4 * gg + 4):
                xsegs += [xrw[:, _TT - 1 - 128 * c - sbl - 16 * s:
                              _TT - 1 - 128 * c - sbl - 16 * s + _L]
                          for s in range(_SEG)]
            xrow4 = jnp.concatenate(xsegs, axis=1)
            hb4 = vwc * xrow4 + c84
            kb4 = _dgd(wk_, hb4)
            for i, sbl in enumerate(range(4 * gg, 4 * gg + 4)):
                kb = kb4[:, 2048 * i: 2048 * (i + 1)]
                tb = 128 * c + sbl
                q8 = jnp.concatenate(
                    [qtok[tb + 16 * s: tb + 16 * s + 1, :]
                     for s in range(_SEG)], axis=0)
                l8 = _dgd(q8, kb)
                rows.append(jnp.sum(l8 * m8, axis=0, keepdims=True))
        lg = jnp.concatenate(rows, axis=0)
        lt = jnp.concatenate(
            [jnp.transpose(lg[:, _L * s: _L * (s + 1)]) for s in range(_SEG)],
            axis=1)

        run = lt
        v1 = jnp.max(run, axis=0, keepdims=True)
        vk = v1
        for _ in range(_TOPK - 1):
            run = jnp.where(run >= vk, -jnp.inf, run)
            vk = jnp.max(run, axis=0, keepdims=True)

        mask = lt >= vk
        ex = jnp.where(mask, jnp.exp(lt - v1), 0.0)
        ssum = jnp.sum(ex, axis=0, keepdims=True)
        wT = ex / ssum

        ytc = yt[:, 128 * c: 128 * (c + 1)]
        mu_parts.append(jnp.sum(wT * ytc, axis=0, keepdims=True))
        w_ref[0, 128 * c: 128 * (c + 1), :] = wT.T

    mu_ar = jnp.concatenate(mu_parts, axis=1)
    mu_ref[0, :, :] = mu_ar + cT


@jax.jit
def kernel(x, lag_embed, val_w, val_b, conv_w, conv_b, ctxp_w, ctxp_b,
           wq, wk, bias_w, bias_b):
    B, T = x.shape

    xf = jnp.pad(x, ((0, 0), (_L, 0)))
    xr = jnp.pad(xf[:, ::-1], ((0, 0), (_L - 1, 1)))
    xf = xf.reshape(B, 1, _L + T)
    xr = xr.reshape(B, 1, 2 * _L + T)

    ec = (lag_embed[1:_L + 1] + val_b[None, :]).T
    c84 = jnp.tile(ec, (1, 4 * _SEG))
    lane = jnp.arange(_SEG * _L, dtype=jnp.int32)[None, :]
    sub = jnp.arange(_SEG, dtype=jnp.int32)[:, None]
    m8 = ((lane >> 8) == sub).astype(jnp.float32)
    vwc = val_w.reshape(_D, 1)
    cw = conv_w.reshape(_D, 9)
    cb = conv_b.reshape(_D, 1)
    cpb = ctxp_b.reshape(_D, 1)
    bb = bias_b.reshape(1, 1)

    nj = T // _TT
    const = lambda shape: pl.BlockSpec(shape, lambda b, j: (0, 0))
    grid_spec = pl.GridSpec(
        grid=(B, nj),
        in_specs=[
            pl.BlockSpec((1, 1, _L + T), lambda b, j: (b, 0, 0)),
            pl.BlockSpec((1, 1, 2 * _L + T), lambda b, j: (b, 0, 0)),
            const((_D, 4 * _SEG * _L)),
            const((_SEG, _SEG * _L)),
            const((_D, 1)),
            const((_D, 9)),
            const((_D, 1)),
            const((_D, _D)),
            const((_D, 1)),
            const((_D, _D)),
            const((_D, _D)),
            const((1, _D)),
            const((1, 1)),
        ],
        out_specs=[
            pl.BlockSpec((1, 1, _TT), lambda b, j: (b, 0, j)),
            pl.BlockSpec((1, _TT, _L), lambda b, j: (b, j, 0)),
        ],
    )
    mu, w = pl.pallas_call(
        _body,
        grid_spec=grid_spec,
        out_shape=[
            jax.ShapeDtypeStruct((B, 1, T), jnp.float32),
            jax.ShapeDtypeStruct((B, T, _L), jnp.float32),
        ],
        compiler_params=pltpu.CompilerParams(
            dimension_semantics=("parallel", "arbitrary"),
        ),
    )(xf, xr, c84, m8, vwc, cw, cb, ctxp_w, cpb, wq, wk, bias_w, bb)
    return (mu.reshape(B, T), w)


# speedup vs baseline: 16.7289x; 1.0496x over previous
import jax
import jax.numpy as jnp
from jax import lax
from jax.experimental import pallas as pl
from jax.experimental.pallas import tpu as pltpu

_L = 256
_D = 32
_TOPK = 16
_TT = 1024
_SEG = 8
_NS = _TT // _SEG


def _dgd(a, b, dims=((1,), (0,))):
    return lax.dot_general(a, b, ((dims), ((), ())),
                           precision=lax.Precision.DEFAULT,
                           preferred_element_type=jnp.float32)


def _bf(a):
    return a.astype(jnp.bfloat16).astype(jnp.float32)


def _body(xf, xr, c84_r, m8_r, vwc_r, cw_r, cb_r, cpw_r, cpb_r, wq_r, wk_r,
          bw_r, bb_r, mu_ref, w_ref):
    j = pl.program_id(1)
    t0 = j * _TT

    c84 = c84_r[:, :]
    m8 = m8_r[:, :]
    vwc, cw, cb = vwc_r[:, :], cw_r[:, :], cb_r[:, :]
    cpw, cpb = cpw_r[:, :], cpb_r[:, :]
    wq_, wk_, bw, bb = wq_r[:, :], wk_r[:, :], bw_r[:, :], bb_r[:, :]

    xw = xf[0, :, pl.ds(t0, _TT + _L)]
    xrw = xr[0, :, pl.ds(2304 - _TT - t0, _TT + _L)]

    xwb = _bf(xw)
    cwb = _bf(cw)
    acc = jnp.broadcast_to(cb, (_D, _TT))
    for k in range(9):
        acc = acc + cwb[:, k][:, None] * xwb[:, _L - 8 + k: _L - 8 + k + _TT]
    ctx2 = _dgd(cpw, acc) + cpb
    qT = _dgd(wq_, ctx2)
    cT = _dgd(bw, ctx2) + bb

    qtok = jnp.transpose(qT)

    base = jnp.broadcast_to(xw, (_L, _TT + _L))
    yt = pltpu.roll(base, _TT + 1, 1, stride=1, stride_axis=0)[:, :_TT]

    mu_parts = []
    for c in range(_TT // 128):
        rows = []
        for gg in range(4):
            xsegs = []
            for sbl in range(4 * gg, 4 * gg + 4):
                xsegs += [xrw[:, _TT - 1 - 128 * c - sbl - 16 * s:
                              _TT - 1 - 128 * c - sbl - 16 * s + _L]
                          for s in range(_SEG)]
            xrow4 = jnp.concatenate(xsegs, axis=1)
            hb4 = vwc * xrow4 + c84
            kb4 = _dgd(wk_, hb4)
            for i, sbl in enumerate(range(4 * gg, 4 * gg + 4)):
                kb = kb4[:, 2048 * i: 2048 * (i + 1)]
                tb = 128 * c + sbl
                q8 = jnp.concatenate(
                    [qtok[tb + 16 * s: tb + 16 * s + 1, :]
                     for s in range(_SEG)], axis=0)
                l8 = _dgd(q8, kb)
                rows.append(jnp.sum(l8 * m8, axis=0, keepdims=True))
        lg = jnp.concatenate(rows, axis=0)
        lt = jnp.concatenate(
            [jnp.transpose(lg[:, _L * s: _L * (s + 1)]) for s in range(_SEG)],
            axis=1)

        run = lt
        v1 = jnp.max(run, axis=0, keepdims=True)
        vk = v1
        for _ in range(_TOPK - 1):
            run = jnp.where(run >= vk, -jnp.inf, run)
            vk = jnp.max(run, axis=0, keepdims=True)

        mask = lt >= vk
        ex = jnp.where(mask, jnp.exp(lt - v1), 0.0)
        inv = 1.0 / jnp.sum(ex, axis=0, keepdims=True)
        wT = ex * inv

        ytc = yt[:, 128 * c: 128 * (c + 1)]
        mu_parts.append(jnp.sum(wT * ytc, axis=0, keepdims=True))
        w_ref[0, 128 * c: 128 * (c + 1), :] = wT.T

    mu_ar = jnp.concatenate(mu_parts, axis=1)
    mu_ref[0, :, :] = mu_ar + cT


@jax.jit
def kernel(x, lag_embed, val_w, val_b, conv_w, conv_b, ctxp_w, ctxp_b,
           wq, wk, bias_w, bias_b):
    B, T = x.shape

    xf = jnp.pad(x, ((0, 0), (_L, 0)))
    xr = jnp.pad(xf[:, ::-1], ((0, 0), (_L - 1, 1)))
    xf = xf.reshape(B, 1, _L + T)
    xr = xr.reshape(B, 1, 2 * _L + T)

    ec = (lag_embed[1:_L + 1] + val_b[None, :]).T
    c84 = jnp.tile(ec, (1, 4 * _SEG))
    lane = jnp.arange(_SEG * _L, dtype=jnp.int32)[None, :]
    sub = jnp.arange(_SEG, dtype=jnp.int32)[:, None]
    m8 = ((lane >> 8) == sub).astype(jnp.float32)
    vwc = val_w.reshape(_D, 1)
    cw = conv_w.reshape(_D, 9)
    cb = conv_b.reshape(_D, 1)
    cpb = ctxp_b.reshape(_D, 1)
    bb = bias_b.reshape(1, 1)

    nj = T // _TT
    const = lambda shape: pl.BlockSpec(shape, lambda b, j: (0, 0))
    grid_spec = pl.GridSpec(
        grid=(B, nj),
        in_specs=[
            pl.BlockSpec((1, 1, _L + T), lambda b, j: (b, 0, 0)),
            pl.BlockSpec((1, 1, 2 * _L + T), lambda b, j: (b, 0, 0)),
            const((_D, 4 * _SEG * _L)),
            const((_SEG, _SEG * _L)),
            const((_D, 1)),
            const((_D, 9)),
            const((_D, 1)),
            const((_D, _D)),
            const((_D, 1)),
            const((_D, _D)),
            const((_D, _D)),
            const((1, _D)),
            const((1, 1)),
        ],
        out_specs=[
            pl.BlockSpec((1, 1, _TT), lambda b, j: (b, 0, j)),
            pl.BlockSpec((1, _TT, _L), lambda b, j: (b, j, 0)),
        ],
    )
    mu, w = pl.pallas_call(
        _body,
        grid_spec=grid_spec,
        out_shape=[
            jax.ShapeDtypeStruct((B, 1, T), jnp.float32),
            jax.ShapeDtypeStruct((B, T, _L), jnp.float32),
        ],
        compiler_params=pltpu.CompilerParams(
            dimension_semantics=("parallel", "arbitrary"),
        ),
    )(xf, xr, c84, m8, vwc, cw, cb, ctxp_w, cpb, wq, wk, bias_w, bb)
    return (mu.reshape(B, T), w)


# speedup vs baseline: 17.0130x; 1.0170x over previous
import jax
import jax.numpy as jnp
from jax import lax
from jax.experimental import pallas as pl
from jax.experimental.pallas import tpu as pltpu

_L = 256
_D = 32
_TOPK = 16
_TT = 2048
_SEG = 8
_NS = _TT // _SEG


def _dgd(a, b, dims=((1,), (0,))):
    return lax.dot_general(a, b, ((dims), ((), ())),
                           precision=lax.Precision.DEFAULT,
                           preferred_element_type=jnp.float32)


def _bf(a):
    return a.astype(jnp.bfloat16).astype(jnp.float32)


def _body(xf, xr, c84_r, m8_r, vwc_r, cw_r, cb_r, cpw_r, cpb_r, wq_r, wk_r,
          bw_r, bb_r, mu_ref, w_ref):
    j = pl.program_id(1)
    t0 = j * _TT

    c84 = c84_r[:, :]
    m8 = m8_r[:, :]
    vwc, cw, cb = vwc_r[:, :], cw_r[:, :], cb_r[:, :]
    cpw, cpb = cpw_r[:, :], cpb_r[:, :]
    wq_, wk_, bw, bb = wq_r[:, :], wk_r[:, :], bw_r[:, :], bb_r[:, :]

    xw = xf[0, :, pl.ds(t0, _TT + _L)]
    xrw = xr[0, :, pl.ds(2304 - _TT - t0, _TT + _L)]

    xwb = _bf(xw)
    cwb = _bf(cw)
    acc = jnp.broadcast_to(cb, (_D, _TT))
    for k in range(9):
        acc = acc + cwb[:, k][:, None] * xwb[:, _L - 8 + k: _L - 8 + k + _TT]
    ctx2 = _dgd(cpw, acc) + cpb
    qT = _dgd(wq_, ctx2)
    cT = _dgd(bw, ctx2) + bb

    qtok = jnp.transpose(qT)

    base = jnp.broadcast_to(xw, (_L, _TT + _L))
    yt = pltpu.roll(base, _TT + 1, 1, stride=1, stride_axis=0)[:, :_TT]

    mu_parts = []
    for c in range(_TT // 128):
        rows = []
        for gg in range(4):
            xsegs = []
            for sbl in range(4 * gg, 4 * gg + 4):
                xsegs += [xrw[:, _TT - 1 - 128 * c - sbl - 16 * s:
                              _TT - 1 - 128 * c - sbl - 16 * s + _L]
                          for s in range(_SEG)]
            xrow4 = jnp.concatenate(xsegs, axis=1)
            hb4 = vwc * xrow4 + c84
            kb4 = _dgd(wk_, hb4)
            for i, sbl in enumerate(range(4 * gg, 4 * gg + 4)):
                kb = kb4[:, 2048 * i: 2048 * (i + 1)]
                tb = 128 * c + sbl
                q8 = jnp.concatenate(
                    [qtok[tb + 16 * s: tb + 16 * s + 1, :]
                     for s in range(_SEG)], axis=0)
                l8 = _dgd(q8, kb)
                rows.append(jnp.sum(l8 * m8, axis=0, keepdims=True))
        lg = jnp.concatenate(rows, axis=0)
        lt = jnp.concatenate(
            [jnp.transpose(lg[:, _L * s: _L * (s + 1)]) for s in range(_SEG)],
            axis=1)

        run = lt
        v1 = jnp.max(run, axis=0, keepdims=True)
        vk = v1
        for _ in range(_TOPK - 1):
            run = jnp.where(run >= vk, -jnp.inf, run)
            vk = jnp.max(run, axis=0, keepdims=True)

        mask = lt >= vk
        ex = jnp.where(mask, jnp.exp(lt - v1), 0.0)
        inv = 1.0 / jnp.sum(ex, axis=0, keepdims=True)
        wT = ex * inv

        ytc = yt[:, 128 * c: 128 * (c + 1)]
        mu_parts.append(jnp.sum(wT * ytc, axis=0, keepdims=True))
        w_ref[0, 128 * c: 128 * (c + 1), :] = wT.T

    mu_ar = jnp.concatenate(mu_parts, axis=1)
    mu_ref[0, :, :] = mu_ar + cT


@jax.jit
def kernel(x, lag_embed, val_w, val_b, conv_w, conv_b, ctxp_w, ctxp_b,
           wq, wk, bias_w, bias_b):
    B, T = x.shape

    xf = jnp.pad(x, ((0, 0), (_L, 0)))
    xr = jnp.pad(xf[:, ::-1], ((0, 0), (_L - 1, 1)))
    xf = xf.reshape(B, 1, _L + T)
    xr = xr.reshape(B, 1, 2 * _L + T)

    ec = (lag_embed[1:_L + 1] + val_b[None, :]).T
    c84 = jnp.tile(ec, (1, 4 * _SEG))
    lane = jnp.arange(_SEG * _L, dtype=jnp.int32)[None, :]
    sub = jnp.arange(_SEG, dtype=jnp.int32)[:, None]
    m8 = ((lane >> 8) == sub).astype(jnp.float32)
    vwc = val_w.reshape(_D, 1)
    cw = conv_w.reshape(_D, 9)
    cb = conv_b.reshape(_D, 1)
    cpb = ctxp_b.reshape(_D, 1)
    bb = bias_b.reshape(1, 1)

    nj = T // _TT
    const = lambda shape: pl.BlockSpec(shape, lambda b, j: (0, 0))
    grid_spec = pl.GridSpec(
        grid=(B, nj),
        in_specs=[
            pl.BlockSpec((1, 1, _L + T), lambda b, j: (b, 0, 0)),
            pl.BlockSpec((1, 1, 2 * _L + T), lambda b, j: (b, 0, 0)),
            const((_D, 4 * _SEG * _L)),
            const((_SEG, _SEG * _L)),
            const((_D, 1)),
            const((_D, 9)),
            const((_D, 1)),
            const((_D, _D)),
            const((_D, 1)),
            const((_D, _D)),
            const((_D, _D)),
            const((1, _D)),
            const((1, 1)),
        ],
        out_specs=[
            pl.BlockSpec((1, 1, _TT), lambda b, j: (b, 0, j)),
            pl.BlockSpec((1, _TT, _L), lambda b, j: (b, j, 0)),
        ],
    )
    mu, w = pl.pallas_call(
        _body,
        grid_spec=grid_spec,
        out_shape=[
            jax.ShapeDtypeStruct((B, 1, T), jnp.float32),
            jax.ShapeDtypeStruct((B, T, _L), jnp.float32),
        ],
        compiler_params=pltpu.CompilerParams(
            dimension_semantics=("parallel", "arbitrary"),
        ),
    )(xf, xr, c84, m8, vwc, cw, cb, ctxp_w, cpb, wq, wk, bias_w, bb)
    return (mu.reshape(B, T), w)
